# Initial kernel scaffold; baseline (speedup 1.0000x reference)
import jax
import jax.numpy as jnp
from jax.experimental import pallas as pl


def kernel(x, edge_index, W1, b1, W2, b2, W3, b3):
    raise NotImplementedError("write your pallas kernel here")


# speedup vs baseline: 7.0318x; 7.0318x over previous
import functools

import jax
import jax.numpy as jnp
from jax import lax
from jax.experimental import pallas as pl
from jax.experimental.pallas import tpu as pltpu
from jax.experimental.pallas import tpu_sc as plsc

N = 10000
D = 128
E = 320000

NC = 2
NS = 16
NW = NC * NS
CH = 128
CPT = 80
EPT = CPT * CH
EPAD = NW * EPT
NROWS = 10240
RPT = NROWS // NS

_f32 = jnp.float32


@functools.lru_cache(maxsize=None)
def _sc_mesh():
    return plsc.VectorSubcoreMesh(core_axis_name="c", subcore_axis_name="s",
                                  num_cores=NC, num_subcores=NS)


def _deg_body(dstr_hbm, out_hbm, acc_sh, dst_v, buf_v):
    cid = lax.axis_index("c")
    sid = lax.axis_index("s")
    wid = cid * NS + sid
    row0 = sid * RPT

    def fill(val):
        def body(r, carry):
            buf_v[r, :] = jnp.full((16,), val, _f32)
            return carry
        lax.fori_loop(0, CH, body, 0)

    fill(0.0)
    for k in range(RPT // CH):
        pltpu.sync_copy(buf_v, acc_sh.at[pl.ds(row0 + k * CH, CH)])
    fill(1.0)

    pltpu.sync_copy(dstr_hbm.at[wid], dst_v)
    plsc.subcore_barrier()

    def chunk(j, carry):
        pltpu.sync_copy(buf_v, acc_sh.at[dst_v.at[j]], add=True)
        return carry
    lax.fori_loop(0, CPT, chunk, 0)

    plsc.subcore_barrier()

    for k in range(RPT // CH):
        pltpu.sync_copy(acc_sh.at[pl.ds(row0 + k * CH, CH)], buf_v)
        pltpu.sync_copy(buf_v, out_hbm.at[cid, pl.ds(row0 + k * CH, CH)])


@functools.lru_cache(maxsize=None)
def _deg_call():
    return pl.kernel(
        _deg_body,
        out_type=jax.ShapeDtypeStruct((NC, NROWS, 16), _f32),
        mesh=_sc_mesh(),
        scratch_types=[
            pltpu.VMEM_SHARED((NROWS, 16), _f32),
            pltpu.VMEM((CPT, CH), jnp.int32),
            pltpu.VMEM((CH, 16), _f32),
        ],
    )


def _agg_body(hs_hbm, srcr_hbm, dstr_hbm, out_hbm, acc_sh, src_v, dst_v,
              rows_v, sem):
    cid = lax.axis_index("c")
    sid = lax.axis_index("s")
    wid = cid * NS + sid
    row0 = sid * RPT

    def zfill(i, carry):
        r = i // 8
        c = (i % 8) * 16
        rows_v[r, pl.ds(c, 16)] = jnp.zeros((16,), _f32)
        return carry
    lax.fori_loop(0, CH * 8, zfill, 0)
    for k in range(RPT // CH):
        pltpu.sync_copy(rows_v, acc_sh.at[pl.ds(row0 + k * CH, CH)])

    pltpu.sync_copy(srcr_hbm.at[wid], src_v)
    pltpu.sync_copy(dstr_hbm.at[wid], dst_v)
    plsc.subcore_barrier()

    def chunk(j, carry):
        pltpu.async_copy(hs_hbm.at[src_v.at[j]], rows_v, sem).wait()
        pltpu.sync_copy(rows_v, acc_sh.at[dst_v.at[j]], add=True)
        return carry
    lax.fori_loop(0, CPT, chunk, 0)

    plsc.subcore_barrier()

    for k in range(RPT // CH):
        pltpu.sync_copy(acc_sh.at[pl.ds(row0 + k * CH, CH)], rows_v)
        pltpu.sync_copy(rows_v, out_hbm.at[cid, pl.ds(row0 + k * CH, CH)])


@functools.lru_cache(maxsize=None)
def _agg_call():
    return pl.kernel(
        _agg_body,
        out_type=jax.ShapeDtypeStruct((NC, NROWS, D), _f32),
        mesh=_sc_mesh(),
        scratch_types=[
            pltpu.VMEM_SHARED((NROWS, D), _f32),
            pltpu.VMEM((CPT, CH), jnp.int32),
            pltpu.VMEM((CPT, CH), jnp.int32),
            pltpu.VMEM((CH, D), _f32),
            pltpu.SemaphoreType.DMA,
        ],
    )


_B = 1000


def _pre_body(d_ref, x_ref, w_ref, dinv_ref, hs_ref):
    deg = d_ref[0, :, 0:1] + d_ref[1, :, 0:1] + 1.0
    dinv = lax.rsqrt(deg)
    dinv_ref[...] = jnp.broadcast_to(dinv, (_B, D))
    hs_ref[...] = jnp.dot(x_ref[...] * dinv, w_ref[...],
                          preferred_element_type=_f32)


_pre_call = pl.pallas_call(
    _pre_body,
    grid=(N // _B,),
    in_specs=[
        pl.BlockSpec((NC, _B, 16), lambda i: (0, i, 0)),
        pl.BlockSpec((_B, D), lambda i: (i, 0)),
        pl.BlockSpec((D, D), lambda i: (0, 0)),
    ],
    out_specs=[
        pl.BlockSpec((_B, D), lambda i: (i, 0)),
        pl.BlockSpec((_B, D), lambda i: (i, 0)),
    ],
    out_shape=[
        jax.ShapeDtypeStruct((N, D), _f32),
        jax.ShapeDtypeStruct((N, D), _f32),
    ],
)


def _mid_body(acc_ref, hs_ref, dinv_ref, b_ref, w_ref, out_ref):
    t = (acc_ref[0] + acc_ref[1] + hs_ref[...]) * dinv_ref[...] + b_ref[...]
    t = jnp.maximum(t, 0.0)
    out_ref[...] = jnp.dot(t * dinv_ref[...], w_ref[...],
                           preferred_element_type=_f32)


_mid_call = pl.pallas_call(
    _mid_body,
    grid=(N // _B,),
    in_specs=[
        pl.BlockSpec((NC, _B, D), lambda i: (0, i, 0)),
        pl.BlockSpec((_B, D), lambda i: (i, 0)),
        pl.BlockSpec((_B, D), lambda i: (i, 0)),
        pl.BlockSpec((1, D), lambda i: (0, 0)),
        pl.BlockSpec((D, D), lambda i: (0, 0)),
    ],
    out_specs=pl.BlockSpec((_B, D), lambda i: (i, 0)),
    out_shape=jax.ShapeDtypeStruct((N, D), _f32),
)


def _fin_body(acc_ref, hs_ref, dinv_ref, b_ref, out_ref):
    out_ref[...] = ((acc_ref[0] + acc_ref[1] + hs_ref[...])
                    * dinv_ref[...] + b_ref[...])


_fin_call = pl.pallas_call(
    _fin_body,
    grid=(N // _B,),
    in_specs=[
        pl.BlockSpec((NC, _B, D), lambda i: (0, i, 0)),
        pl.BlockSpec((_B, D), lambda i: (i, 0)),
        pl.BlockSpec((_B, D), lambda i: (i, 0)),
        pl.BlockSpec((1, D), lambda i: (0, 0)),
    ],
    out_specs=pl.BlockSpec((_B, D), lambda i: (i, 0)),
    out_shape=jax.ShapeDtypeStruct((N, D), _f32),
)


def kernel(x, edge_index, W1, b1, W2, b2, W3, b3):
    ei = edge_index.astype(jnp.int32)
    pad = EPAD - E
    src = jnp.concatenate([ei[0], jnp.zeros((pad,), jnp.int32)])
    dst = jnp.concatenate([ei[1], jnp.full((pad,), N, jnp.int32)])
    srcr = src.reshape(NW, CPT, CH)
    dstr = dst.reshape(NW, CPT, CH)

    degs = _deg_call()(dstr)
    dinv, hs = _pre_call(degs[:, :N], x, W1)

    agg = _agg_call()
    acc = agg(hs, srcr, dstr)
    hs = _mid_call(acc[:, :N], hs, dinv, b1.reshape(1, D), W2)
    acc = agg(hs, srcr, dstr)
    hs = _mid_call(acc[:, :N], hs, dinv, b2.reshape(1, D), W3)
    acc = agg(hs, srcr, dstr)
    return _fin_call(acc[:, :N], hs, dinv, b3.reshape(1, D))
